# baseline (speedup 1.0000x reference)
import functools

import jax
import jax.numpy as jnp
from jax import lax
from jax.experimental import pallas as pl
from jax.experimental.pallas import tpu as pltpu
from jax.experimental.pallas import tpu_sc as plsc

NUM_WORDS = 1000000
EMB = 32
BATCH = 16384
HIST = 50
NW = 32
ELEMS_PER_W = BATCH // NW
GROUP = 16
NGROUPS = ELEMS_PER_W // GROUP
ITERS = NGROUPS // 2
TR = EMB // 8
CB = BATCH // 128


def _gather_body(idx_hbm, table_hbm, out_hbm, idx_v, buf_a, buf_b,
                 stage_a, stage_b, gs_a, gs_b, wb_a, wb_b):
    wid = lax.axis_index("s") * 2 + lax.axis_index("c")
    base = wid * ELEMS_PER_W
    pltpu.sync_copy(idx_hbm.at[wid], idx_v)

    bufs = (buf_a, buf_b)
    stages = (stage_a, stage_b)
    g_sems = (gs_a, gs_b)
    wb_sems = (wb_a, wb_b)

    e_iota = lax.iota(jnp.int32, 16) * HIST

    def fire(g, half):
        for j in range(GROUP):
            pltpu.async_copy(
                table_hbm.at[idx_v.at[g * GROUP + j]],
                bufs[half].at[pl.ds(j * HIST, HIST)],
                g_sems[half],
            )

    fire(0, 0)

    def step(i, carry):
        for half in range(2):
            g = 2 * i + half
            buf, stage = bufs[half], stages[half]

            pltpu.make_async_copy(
                table_hbm.at[pl.ds(0, GROUP * HIST)], buf,
                g_sems[half]).wait()

            @pl.when(g + 1 < NGROUPS)
            def _():
                fire(g + 1, 1 - half)

            @pl.when(i >= 1)
            def _():
                pltpu.make_async_copy(
                    out_hbm.at[:, :, 0, :, pl.ds(0, GROUP)], stage,
                    wb_sems[half]).wait()

            def transpose_t(t, c):
                row = e_iota + t
                for d in range(EMB):
                    v = plsc.load_gather(
                        buf, [row, jnp.full((16,), d, jnp.int32)])
                    stage[t, d // 8, d % 8, :] = v
                return c

            lax.fori_loop(0, HIST, transpose_t, 0)

            b0 = base + g * GROUP
            cb = b0 // 128
            l0 = b0 % 128
            pltpu.async_copy(
                stage, out_hbm.at[:, :, cb, :, pl.ds(l0, GROUP)],
                wb_sems[half])
        return carry

    lax.fori_loop(0, ITERS, step, 0)

    for half in range(2):
        pltpu.make_async_copy(
            out_hbm.at[:, :, 0, :, pl.ds(0, GROUP)], stages[half],
            wb_sems[half]).wait()


@jax.jit
def kernel(indices, embedding_weight):
    idx = indices.astype(jnp.int32).reshape(NW, ELEMS_PER_W, HIST)
    mesh = plsc.VectorSubcoreMesh(core_axis_name="c", subcore_axis_name="s")
    out5 = pl.kernel(
        _gather_body,
        mesh=mesh,
        out_type=jax.ShapeDtypeStruct((HIST, TR, CB, 8, 128), jnp.float32),
        compiler_params=pltpu.CompilerParams(
            use_tc_tiling_on_sc=False, needs_layout_passes=False),
        scratch_types=[
            pltpu.VMEM((ELEMS_PER_W, HIST), jnp.int32),
            pltpu.VMEM((GROUP * HIST, EMB), jnp.float32),
            pltpu.VMEM((GROUP * HIST, EMB), jnp.float32),
            pltpu.VMEM((HIST, TR, 8, GROUP), jnp.float32),
            pltpu.VMEM((HIST, TR, 8, GROUP), jnp.float32),
            pltpu.SemaphoreType.DMA,
            pltpu.SemaphoreType.DMA,
            pltpu.SemaphoreType.DMA,
            pltpu.SemaphoreType.DMA,
        ],
    )(idx, embedding_weight)
    return out5.transpose((2, 4, 0, 1, 3)).reshape(BATCH, HIST, EMB)
